# baseline (speedup 1.0000x reference)
import jax
import jax.numpy as jnp
from jax.experimental import pallas as pl
from jax.experimental.pallas import tpu as pltpu

B, T, H, E, C = 4, 2048, 2048, 16, 256
TB = 256
NJ = T // TB
EPS = 1e-9
THRESH = 0.2
CHUNK = 256
NCH = T // CHUNK
XC = 512
NXC = T // XC


def _route_body(x_ref, w_ref, probs_ref, meta_ref, logits_scr):
    j = pl.program_id(1)
    wg = w_ref[...]
    logits_scr[:, pl.ds(j * XC, XC)] = jax.lax.dot_general(
        wg, x_ref[0], (((1,), (1,)), ((), ())),
        preferred_element_type=jnp.float32)

    @pl.when(j == NXC - 1)
    def _finish():
        _route_finish(probs_ref, meta_ref, logits_scr)


def _route_finish(probs_ref, meta_ref, logits_scr):
    logits = logits_scr[...]
    m = jnp.max(logits, axis=0, keepdims=True)
    ex = jnp.exp(logits - m)
    sm = ex / jnp.sum(ex, axis=0, keepdims=True)

    row = jax.lax.broadcasted_iota(jnp.int32, (E, T), 0)
    v0 = jnp.max(sm, axis=0, keepdims=True)
    e0 = jnp.min(jnp.where(sm == v0, row, E), axis=0, keepdims=True)
    sm1 = jnp.where(row == e0, -1.0, sm)
    v1 = jnp.max(sm1, axis=0, keepdims=True)
    e1 = jnp.min(jnp.where(sm1 == v1, row, E), axis=0, keepdims=True)

    denom = jnp.maximum(v0 + v1, EPS)
    g0 = v0 / denom
    g1 = v1 / denom

    route1 = probs_ref[0] < g1 / jnp.float32(THRESH)
    mask0 = (row == e0).astype(jnp.float32)
    mask1 = jnp.where(route1, (row == e1).astype(jnp.float32), 0.0)

    r = jax.lax.broadcasted_iota(jnp.int32, (CHUNK, CHUNK), 0)
    c = jax.lax.broadcasted_iota(jnp.int32, (CHUNK, CHUNK), 1)
    sut = (r < c).astype(jnp.float32)
    mcat = jnp.concatenate([mask0, mask1], axis=0)
    carry = jnp.zeros((2 * E, 1), dtype=jnp.float32)
    chunks = []
    for i in range(NCH):
        mc = mcat[:, i * CHUNK:(i + 1) * CHUNK]
        chunks.append(jnp.dot(mc, sut, preferred_element_type=jnp.float32)
                      + carry)
        carry = carry + jnp.sum(mc, axis=1, keepdims=True)
    rankcat = jnp.concatenate(chunks, axis=1)
    rank0 = rankcat[:E]
    rank1 = rankcat[E:]
    count0 = jnp.minimum(carry[:E], float(C))

    p0 = jnp.sum(rank0 * mask0, axis=0, keepdims=True)
    p1 = jnp.sum((rank1 + count0) * mask1, axis=0, keepdims=True)
    keep0 = p0 < float(C)
    keep1 = route1 & (p1 < float(C))

    tgt0 = jnp.where(keep0, e0.astype(jnp.float32) * float(C) + p0, -1.0)
    tgt1 = jnp.where(keep1, e1.astype(jnp.float32) * float(C) + p1, -1.0)
    g0k = jnp.where(keep0, g0, 0.0)
    g1k = jnp.where(keep1, g1, 0.0)
    meta4 = jnp.concatenate([tgt0, tgt1, g0k, g1k], axis=0)
    meta_ref[0] = jnp.transpose(meta4, (1, 0))


def _expand_body(meta_ref, flat_ref, disp_ref, comb_ref):
    rows = meta_ref[0]
    t0 = rows[:, 0:1].reshape(TB, 1, 1).astype(jnp.int32)
    t1 = rows[:, 1:2].reshape(TB, 1, 1).astype(jnp.int32)
    g0 = rows[:, 2:3].reshape(TB, 1, 1)
    g1 = rows[:, 3:4].reshape(TB, 1, 1)
    flat = flat_ref[...]
    m0 = flat == t0
    m1 = flat == t1
    comb = jnp.where(m0, g0, 0.0) + jnp.where(m1, g1, 0.0)
    disp_ref[0] = (comb != 0.0).astype(jnp.float32)
    comb_ref[0] = comb


def kernel(x, w_g, probs):
    probs1 = probs[1].reshape(B, 1, T)
    meta = pl.pallas_call(
        _route_body,
        grid=(B, NXC),
        in_specs=[
            pl.BlockSpec((1, XC, H), lambda b, j: (b, j, 0)),
            pl.BlockSpec((E, H), lambda b, j: (0, 0)),
            pl.BlockSpec((1, 1, T), lambda b, j: (b, 0, 0)),
        ],
        out_specs=pl.BlockSpec((1, T, 4), lambda b, j: (b, 0, 0)),
        out_shape=jax.ShapeDtypeStruct((B, T, 4), jnp.float32),
        scratch_shapes=[pltpu.VMEM((E, T), jnp.float32)],
    )(x, w_g, probs1)

    fe = jax.lax.broadcasted_iota(jnp.int32, (1, E, C), 1)
    fc = jax.lax.broadcasted_iota(jnp.int32, (1, E, C), 2)
    flat = fe * C + fc

    disp, comb = pl.pallas_call(
        _expand_body,
        grid=(B, NJ),
        in_specs=[
            pl.BlockSpec((1, TB, 4), lambda b, j: (b, j, 0)),
            pl.BlockSpec((1, E, C), lambda b, j: (0, 0, 0)),
        ],
        out_specs=[
            pl.BlockSpec((1, TB, E, C), lambda b, j: (b, j, 0, 0)),
            pl.BlockSpec((1, TB, E, C), lambda b, j: (b, j, 0, 0)),
        ],
        out_shape=[
            jax.ShapeDtypeStruct((B, T, E, C), jnp.float32),
            jax.ShapeDtypeStruct((B, T, E, C), jnp.float32),
        ],
    )(meta, flat)
    return disp, comb


# speedup vs baseline: 1.0153x; 1.0153x over previous
import jax
import jax.numpy as jnp
from jax.experimental import pallas as pl

B, T, H, E, C = 4, 2048, 2048, 16, 256
TB = 256
NJ = T // TB
EPS = 1e-9
THRESH = 0.2
CHUNK = 256
NCH = T // CHUNK


def _route_body(x_ref, w_ref, probs_ref, meta_ref):
    xb = x_ref[0]
    wg = w_ref[...]
    logits = jax.lax.dot_general(wg, xb, (((1,), (1,)), ((), ())),
                                 preferred_element_type=jnp.float32)
    m = jnp.max(logits, axis=0, keepdims=True)
    ex = jnp.exp(logits - m)
    sm = ex / jnp.sum(ex, axis=0, keepdims=True)

    row = jax.lax.broadcasted_iota(jnp.int32, (E, T), 0)
    v0 = jnp.max(sm, axis=0, keepdims=True)
    e0 = jnp.min(jnp.where(sm == v0, row, E), axis=0, keepdims=True)
    sm1 = jnp.where(row == e0, -1.0, sm)
    v1 = jnp.max(sm1, axis=0, keepdims=True)
    e1 = jnp.min(jnp.where(sm1 == v1, row, E), axis=0, keepdims=True)

    denom = jnp.maximum(v0 + v1, EPS)
    g0 = v0 / denom
    g1 = v1 / denom

    route1 = probs_ref[0] < g1 / jnp.float32(THRESH)
    mask0 = (row == e0).astype(jnp.float32)
    mask1 = jnp.where(route1, (row == e1).astype(jnp.float32), 0.0)

    r = jax.lax.broadcasted_iota(jnp.int32, (CHUNK, CHUNK), 0)
    c = jax.lax.broadcasted_iota(jnp.int32, (CHUNK, CHUNK), 1)
    sut = (r < c).astype(jnp.float32)
    mcat = jnp.concatenate([mask0, mask1], axis=0)
    carry = jnp.zeros((2 * E, 1), dtype=jnp.float32)
    chunks = []
    for i in range(NCH):
        mc = mcat[:, i * CHUNK:(i + 1) * CHUNK]
        chunks.append(jnp.dot(mc, sut, preferred_element_type=jnp.float32)
                      + carry)
        carry = carry + jnp.sum(mc, axis=1, keepdims=True)
    rankcat = jnp.concatenate(chunks, axis=1)
    rank0 = rankcat[:E]
    rank1 = rankcat[E:]
    count0 = jnp.minimum(carry[:E], float(C))

    p0 = jnp.sum(rank0 * mask0, axis=0, keepdims=True)
    p1 = jnp.sum((rank1 + count0) * mask1, axis=0, keepdims=True)
    keep0 = p0 < float(C)
    keep1 = route1 & (p1 < float(C))

    tgt0 = jnp.where(keep0, e0.astype(jnp.float32) * float(C) + p0, -1.0)
    tgt1 = jnp.where(keep1, e1.astype(jnp.float32) * float(C) + p1, -1.0)
    g0k = jnp.where(keep0, g0, 0.0)
    g1k = jnp.where(keep1, g1, 0.0)
    meta4 = jnp.concatenate([tgt0, tgt1, g0k, g1k], axis=0)
    meta_ref[0] = jnp.transpose(meta4, (1, 0))


def _expand_body(meta_ref, flat_ref, disp_ref, comb_ref):
    rows = meta_ref[0]
    t0 = rows[:, 0:1].reshape(TB, 1, 1).astype(jnp.int32)
    t1 = rows[:, 1:2].reshape(TB, 1, 1).astype(jnp.int32)
    g0 = rows[:, 2:3].reshape(TB, 1, 1)
    g1 = rows[:, 3:4].reshape(TB, 1, 1)
    flat = flat_ref[...]
    m0 = flat == t0
    m1 = flat == t1
    comb = jnp.where(m0, g0, 0.0) + jnp.where(m1, g1, 0.0)
    disp_ref[0] = (comb != 0.0).astype(jnp.float32)
    comb_ref[0] = comb


def kernel(x, w_g, probs):
    probs1 = probs[1].reshape(B, 1, T)
    meta = pl.pallas_call(
        _route_body,
        grid=(B,),
        in_specs=[
            pl.BlockSpec((1, T, H), lambda b: (b, 0, 0)),
            pl.BlockSpec((E, H), lambda b: (0, 0)),
            pl.BlockSpec((1, 1, T), lambda b: (b, 0, 0)),
        ],
        out_specs=pl.BlockSpec((1, T, 4), lambda b: (b, 0, 0)),
        out_shape=jax.ShapeDtypeStruct((B, T, 4), jnp.float32),
    )(x, w_g, probs1)

    fe = jax.lax.broadcasted_iota(jnp.int32, (1, E, C), 1)
    fc = jax.lax.broadcasted_iota(jnp.int32, (1, E, C), 2)
    flat = fe * C + fc

    disp, comb = pl.pallas_call(
        _expand_body,
        grid=(B, NJ),
        in_specs=[
            pl.BlockSpec((1, TB, 4), lambda b, j: (b, j, 0)),
            pl.BlockSpec((1, E, C), lambda b, j: (0, 0, 0)),
        ],
        out_specs=[
            pl.BlockSpec((1, TB, E, C), lambda b, j: (b, j, 0, 0)),
            pl.BlockSpec((1, TB, E, C), lambda b, j: (b, j, 0, 0)),
        ],
        out_shape=[
            jax.ShapeDtypeStruct((B, T, E, C), jnp.float32),
            jax.ShapeDtypeStruct((B, T, E, C), jnp.float32),
        ],
    )(meta, flat)
    return disp, comb
